# baseline (speedup 1.0000x reference)
import functools

import numpy as np
import jax
import jax.numpy as jnp
from jax import lax
from jax.experimental import pallas as pl
from jax.experimental.pallas import tpu as pltpu
from jax.experimental.pallas import tpu_sc as plsc

_L = 32
_AG = 32
_B = 4096
_N = _AG + _L
_C = _N * (_N - 1) // 2

_G = 64
_triu = np.stack(np.triu_indices(_N, k=1)).astype(np.int32)
_TB_NP = (np.tile(_triu, (1, _G))
          + (np.arange(_G, dtype=np.int32).repeat(_C) * _N)[None, :])


_BB = 64
_OBS = 4 + 2 * _L + 2 * (_AG - 1) + (_AG - 1)
_SD = _OBS + 3


def _build_selectors():
    m1 = np.zeros((_SD, _N * 8), np.float32)
    m2 = np.zeros((_SD, _N * 8), np.float32)
    m3 = np.zeros((_SD, _N * 8), np.float32)
    one_c = _OBS + 2
    inv_c = (_OBS, _OBS + 1)
    m1[2, 0] = 1.0
    m1[3, 1] = 1.0
    m1[0, 2] = 1.0
    m1[1, 3] = 1.0
    m1[one_c, 6] = 2.0
    for n in range(1, _N):
        w = 8 * n
        if n <= _L:
            base = 4 + 2 * (n - 1)
        else:
            base = 4 + 2 * _L + 2 * (n - 1 - _L)
        for d in range(2):
            m1[2 + d, w + d] = 1.0
            m1[base + d, w + d] = 1.0
            m1[base + d, w + 2 + d] = 1.0
            m2[base + d, w + 4 + d] = 1.0
            m3[inv_c[d], w + 4 + d] = 1.0
        if n > _L:
            j = n - 1 - _L
            m1[one_c, w + 6] = 1.0
            m1[4 + 2 * _L + 2 * (_AG - 1) + j, w + 7] = 1.0
    return m1, m2, m3


_M1_NP, _M2_NP, _M3_NP = _build_selectors()


def _tc_body(tb_ref, obs_ref, m1_ref, m2_ref, m3_ref, ei_ref, x_ref, ea_ref):
    off = pl.program_id(0) * (_G * _N)
    ei_ref[...] = tb_ref[...] + off

    obs = obs_ref[...]
    vel = obs[:, 0:2]
    inv = 1.0 / (0.001 + vel)
    one = jnp.ones((_BB, 1), jnp.float32)
    s = jnp.concatenate([obs, inv, one], axis=1)
    dn = (((1,), (0,)), ((), ()))
    hi = jax.lax.Precision.HIGHEST
    a = jax.lax.dot_general(s, m1_ref[...], dn, precision=hi)
    b = jax.lax.dot_general(s, m2_ref[...], dn, precision=hi)
    c = jax.lax.dot_general(s, m3_ref[...], dn, precision=hi)
    x512 = a + b * c
    x_ref[...] = x512.reshape(_BB, _N, 8)

    d1 = x512[:, 0:8] - x512[:, 8:16]
    d2 = x512[:, 0:8] - x512[:, 16:24]
    e1 = jnp.sum(d1 * d1, axis=1, keepdims=True)
    e2 = jnp.sum(d2 * d2, axis=1, keepdims=True)
    ea_ref[...] = jnp.sqrt(jnp.concatenate([e1, e2], axis=1))


_tc_call = pl.pallas_call(
    _tc_body,
    grid=(_B // _BB,),
    in_specs=[
        pl.BlockSpec((2, _G * _C), lambda i: (0, 0)),
        pl.BlockSpec((_BB, _OBS), lambda i: (i, 0)),
        pl.BlockSpec((_SD, _N * 8), lambda i: (0, 0)),
        pl.BlockSpec((_SD, _N * 8), lambda i: (0, 0)),
        pl.BlockSpec((_SD, _N * 8), lambda i: (0, 0)),
    ],
    out_specs=[
        pl.BlockSpec((2, _G * _C), lambda i: (0, i)),
        pl.BlockSpec((_BB, _N, 8), lambda i: (i, 0, 0)),
        pl.BlockSpec((_BB, 2), lambda i: (i, 0)),
    ],
    out_shape=[
        jax.ShapeDtypeStruct((2, _B * _C), jnp.int32),
        jax.ShapeDtypeStruct((_B, _N, 8), jnp.float32),
        jax.ShapeDtypeStruct((_B, 2), jnp.float32),
    ],
)


_NW = 32
_BPW = _B // _NW


def _sc_bv_body(bv_hbm, buf):
    core = lax.axis_index("c")
    sub = lax.axis_index("s")
    wid = sub * 2 + core
    i0 = wid * _BPW
    zero16 = lax.broadcasted_iota(jnp.int32, (16,), 0) * 0
    for k in range(_BPW):
        val = zero16 + (i0 + k)
        for v in range(_N // 16):
            buf[pl.ds(k * _N + v * 16, 16)] = val
    pltpu.sync_copy(buf, bv_hbm.at[pl.ds(i0 * _N, _BPW * _N)])


@functools.lru_cache(maxsize=None)
def _sc_bv_call():
    return pl.kernel(
        _sc_bv_body,
        out_type=jax.ShapeDtypeStruct((_B * _N,), jnp.int32),
        mesh=plsc.VectorSubcoreMesh(core_axis_name="c", subcore_axis_name="s"),
        scratch_types=[pltpu.VMEM((_BPW * _N,), jnp.int32)],
    )


def kernel(batch_observations):
    obs = batch_observations

    ei, x3, ea = _tc_call(jnp.asarray(_TB_NP), obs, jnp.asarray(_M1_NP),
                          jnp.asarray(_M2_NP), jnp.asarray(_M3_NP))
    bv = _sc_bv_call()()

    return (x3.reshape(_B * _N, 8),
            ei,
            ea.reshape(-1),
            bv)


# speedup vs baseline: 1.1149x; 1.1149x over previous
import functools

import numpy as np
import jax
import jax.numpy as jnp
from jax import lax
from jax.experimental import pallas as pl
from jax.experimental.pallas import tpu as pltpu
from jax.experimental.pallas import tpu_sc as plsc

_L = 32
_AG = 32
_B = 4096
_N = _AG + _L
_C = _N * (_N - 1) // 2

_G = 64
_triu = np.stack(np.triu_indices(_N, k=1)).astype(np.int32)
_TB_NP = (np.tile(_triu, (1, _G))
          + (np.arange(_G, dtype=np.int32).repeat(_C) * _N)[None, :])


def _ei_body(tb_ref, ei_ref):
    off = pl.program_id(0) * (_G * _N)
    ei_ref[...] = tb_ref[...] + off


_ei_call = pl.pallas_call(
    _ei_body,
    grid=(_B // _G,),
    in_specs=[pl.BlockSpec((2, _G * _C), lambda i: (0, 0))],
    out_specs=pl.BlockSpec((2, _G * _C), lambda i: (0, i)),
    out_shape=jax.ShapeDtypeStruct((2, _B * _C), jnp.int32),
)


_BB = 128
_OBS = 4 + 2 * _L + 2 * (_AG - 1) + (_AG - 1)
_SD = _OBS + 3


def _build_selectors():
    m1 = np.zeros((_SD, _N * 8), np.float32)
    m2 = np.zeros((_SD, _N * 8), np.float32)
    m3 = np.zeros((_SD, _N * 8), np.float32)
    one_c = _OBS + 2
    inv_c = (_OBS, _OBS + 1)
    m1[2, 0] = 1.0
    m1[3, 1] = 1.0
    m1[0, 2] = 1.0
    m1[1, 3] = 1.0
    m1[one_c, 6] = 2.0
    for n in range(1, _N):
        w = 8 * n
        if n <= _L:
            base = 4 + 2 * (n - 1)
        else:
            base = 4 + 2 * _L + 2 * (n - 1 - _L)
        for d in range(2):
            m1[2 + d, w + d] = 1.0
            m1[base + d, w + d] = 1.0
            m1[base + d, w + 2 + d] = 1.0
            m2[base + d, w + 4 + d] = 1.0
            m3[inv_c[d], w + 4 + d] = 1.0
        if n > _L:
            j = n - 1 - _L
            m1[one_c, w + 6] = 1.0
            m1[4 + 2 * _L + 2 * (_AG - 1) + j, w + 7] = 1.0
    return m1, m2, m3


_M1_NP, _M2_NP, _M3_NP = _build_selectors()


def _tc_body(obs_ref, m1_ref, m2_ref, m3_ref, x_ref, ea_ref):
    obs = obs_ref[...]
    vel = obs[:, 0:2]
    inv = 1.0 / (0.001 + vel)
    one = jnp.ones((_BB, 1), jnp.float32)
    s = jnp.concatenate([obs, inv, one], axis=1)
    dn = (((1,), (0,)), ((), ()))
    hi = jax.lax.Precision.HIGHEST
    a = jax.lax.dot_general(s, m1_ref[...], dn, precision=hi)
    b = jax.lax.dot_general(s, m2_ref[...], dn, precision=hi)
    c = jax.lax.dot_general(s, m3_ref[...], dn, precision=hi)
    x512 = a + b * c
    x_ref[...] = x512.reshape(_BB, _N, 8)

    d1 = x512[:, 0:8] - x512[:, 8:16]
    d2 = x512[:, 0:8] - x512[:, 16:24]
    e1 = jnp.sum(d1 * d1, axis=1, keepdims=True)
    e2 = jnp.sum(d2 * d2, axis=1, keepdims=True)
    ea_ref[...] = jnp.sqrt(jnp.concatenate([e1, e2], axis=1))


_tc_call = pl.pallas_call(
    _tc_body,
    grid=(_B // _BB,),
    in_specs=[
        pl.BlockSpec((_BB, _OBS), lambda i: (i, 0)),
        pl.BlockSpec((_SD, _N * 8), lambda i: (0, 0)),
        pl.BlockSpec((_SD, _N * 8), lambda i: (0, 0)),
        pl.BlockSpec((_SD, _N * 8), lambda i: (0, 0)),
    ],
    out_specs=[
        pl.BlockSpec((_BB, _N, 8), lambda i: (i, 0, 0)),
        pl.BlockSpec((_BB, 2), lambda i: (i, 0)),
    ],
    out_shape=[
        jax.ShapeDtypeStruct((_B, _N, 8), jnp.float32),
        jax.ShapeDtypeStruct((_B, 2), jnp.float32),
    ],
)


_NW = 32
_BPW = _B // _NW


def _sc_bv_body(bv_hbm, buf):
    core = lax.axis_index("c")
    sub = lax.axis_index("s")
    wid = sub * 2 + core
    i0 = wid * _BPW
    zero16 = lax.broadcasted_iota(jnp.int32, (16,), 0) * 0
    for k in range(_BPW):
        val = zero16 + (i0 + k)
        for v in range(_N // 16):
            buf[pl.ds(k * _N + v * 16, 16)] = val
    pltpu.sync_copy(buf, bv_hbm.at[pl.ds(i0 * _N, _BPW * _N)])


@functools.lru_cache(maxsize=None)
def _sc_bv_call():
    return pl.kernel(
        _sc_bv_body,
        out_type=jax.ShapeDtypeStruct((_B * _N,), jnp.int32),
        mesh=plsc.VectorSubcoreMesh(core_axis_name="c", subcore_axis_name="s"),
        scratch_types=[pltpu.VMEM((_BPW * _N,), jnp.int32)],
    )


def kernel(batch_observations):
    obs = batch_observations

    ei = _ei_call(jnp.asarray(_TB_NP))
    x3, ea = _tc_call(obs, jnp.asarray(_M1_NP), jnp.asarray(_M2_NP),
                      jnp.asarray(_M3_NP))
    bv = _sc_bv_call()()

    return (x3.reshape(_B * _N, 8),
            ei,
            ea.reshape(-1),
            bv)


# speedup vs baseline: 1.1341x; 1.0173x over previous
import functools

import numpy as np
import jax
import jax.numpy as jnp
from jax import lax
from jax.experimental import pallas as pl
from jax.experimental.pallas import tpu as pltpu
from jax.experimental.pallas import tpu_sc as plsc

_L = 32
_AG = 32
_B = 4096
_N = _AG + _L
_C = _N * (_N - 1) // 2

_G = 64
_triu = np.stack(np.triu_indices(_N, k=1)).astype(np.int32)
_TB_NP = (np.tile(_triu, (1, _G))
          + (np.arange(_G, dtype=np.int32).repeat(_C) * _N)[None, :])


def _ei_body(tb_ref, ei_ref):
    off = pl.program_id(0) * (_G * _N)
    ei_ref[...] = tb_ref[...] + off


_ei_call = pl.pallas_call(
    _ei_body,
    grid=(_B // _G,),
    in_specs=[pl.BlockSpec((2, _G * _C), lambda i: (0, 0))],
    out_specs=pl.BlockSpec((2, _G * _C), lambda i: (0, i)),
    out_shape=jax.ShapeDtypeStruct((2, _B * _C), jnp.int32),
)


_BB = 256
_OBS = 4 + 2 * _L + 2 * (_AG - 1) + (_AG - 1)
_SD = _OBS + 3


def _build_selectors():
    m1 = np.zeros((_SD, _N * 8), np.float32)
    m2 = np.zeros((_SD, _N * 8), np.float32)
    m3 = np.zeros((_SD, _N * 8), np.float32)
    one_c = _OBS + 2
    inv_c = (_OBS, _OBS + 1)
    m1[2, 0] = 1.0
    m1[3, 1] = 1.0
    m1[0, 2] = 1.0
    m1[1, 3] = 1.0
    m1[one_c, 6] = 2.0
    for n in range(1, _N):
        w = 8 * n
        if n <= _L:
            base = 4 + 2 * (n - 1)
        else:
            base = 4 + 2 * _L + 2 * (n - 1 - _L)
        for d in range(2):
            m1[2 + d, w + d] = 1.0
            m1[base + d, w + d] = 1.0
            m1[base + d, w + 2 + d] = 1.0
            m2[base + d, w + 4 + d] = 1.0
            m3[inv_c[d], w + 4 + d] = 1.0
        if n > _L:
            j = n - 1 - _L
            m1[one_c, w + 6] = 1.0
            m1[4 + 2 * _L + 2 * (_AG - 1) + j, w + 7] = 1.0
    return m1, m2, m3


_M1_NP, _M2_NP, _M3_NP = _build_selectors()


def _tc_body(obs_ref, m1_ref, m2_ref, m3_ref, x_ref, ea_ref):
    obs = obs_ref[...]
    vel = obs[:, 0:2]
    inv = 1.0 / (0.001 + vel)
    one = jnp.ones((_BB, 1), jnp.float32)
    s = jnp.concatenate([obs, inv, one], axis=1)
    dn = (((1,), (0,)), ((), ()))
    hi = jax.lax.Precision.HIGHEST
    a = jax.lax.dot_general(s, m1_ref[...], dn, precision=hi)
    b = jax.lax.dot_general(s, m2_ref[...], dn, precision=hi)
    c = jax.lax.dot_general(s, m3_ref[...], dn, precision=hi)
    x512 = a + b * c
    x_ref[...] = x512.reshape(_BB, _N, 8)

    d1 = x512[:, 0:8] - x512[:, 8:16]
    d2 = x512[:, 0:8] - x512[:, 16:24]
    e1 = jnp.sum(d1 * d1, axis=1, keepdims=True)
    e2 = jnp.sum(d2 * d2, axis=1, keepdims=True)
    ea_ref[...] = jnp.sqrt(jnp.concatenate([e1, e2], axis=1))


_tc_call = pl.pallas_call(
    _tc_body,
    grid=(_B // _BB,),
    in_specs=[
        pl.BlockSpec((_BB, _OBS), lambda i: (i, 0)),
        pl.BlockSpec((_SD, _N * 8), lambda i: (0, 0)),
        pl.BlockSpec((_SD, _N * 8), lambda i: (0, 0)),
        pl.BlockSpec((_SD, _N * 8), lambda i: (0, 0)),
    ],
    out_specs=[
        pl.BlockSpec((_BB, _N, 8), lambda i: (i, 0, 0)),
        pl.BlockSpec((_BB, 2), lambda i: (i, 0)),
    ],
    out_shape=[
        jax.ShapeDtypeStruct((_B, _N, 8), jnp.float32),
        jax.ShapeDtypeStruct((_B, 2), jnp.float32),
    ],
)


_NW = 32
_BPW = _B // _NW


def _sc_bv_body(bv_hbm, buf):
    core = lax.axis_index("c")
    sub = lax.axis_index("s")
    wid = sub * 2 + core
    i0 = wid * _BPW
    zero16 = lax.broadcasted_iota(jnp.int32, (16,), 0) * 0
    for k in range(_BPW):
        val = zero16 + (i0 + k)
        for v in range(_N // 16):
            buf[pl.ds(k * _N + v * 16, 16)] = val
    pltpu.sync_copy(buf, bv_hbm.at[pl.ds(i0 * _N, _BPW * _N)])


@functools.lru_cache(maxsize=None)
def _sc_bv_call():
    return pl.kernel(
        _sc_bv_body,
        out_type=jax.ShapeDtypeStruct((_B * _N,), jnp.int32),
        mesh=plsc.VectorSubcoreMesh(core_axis_name="c", subcore_axis_name="s"),
        scratch_types=[pltpu.VMEM((_BPW * _N,), jnp.int32)],
    )


def kernel(batch_observations):
    obs = batch_observations

    ei = _ei_call(jnp.asarray(_TB_NP))
    x3, ea = _tc_call(obs, jnp.asarray(_M1_NP), jnp.asarray(_M2_NP),
                      jnp.asarray(_M3_NP))
    bv = _sc_bv_call()()

    return (x3.reshape(_B * _N, 8),
            ei,
            ea.reshape(-1),
            bv)
